# baseline (speedup 1.0000x reference)
import functools

import jax
import jax.numpy as jnp
from jax import lax
from jax.experimental import pallas as pl
from jax.experimental.pallas import tpu as pltpu
from jax.experimental.pallas import tpu_sc as plsc

D = 64
B = 16384 * 26
NC, NS = 2, 16
NW = NC * NS
CHUNK = 128
ROWS_PER_W = B // NW
NCHUNK = ROWS_PER_W // CHUNK
NBUF = 4

_mesh = plsc.VectorSubcoreMesh(core_axis_name="c", subcore_axis_name="s")


@functools.partial(
    pl.kernel,
    out_type=jax.ShapeDtypeStruct((B, D), jnp.float32),
    mesh=_mesh,
    scratch_types=(
        [pltpu.VMEM((NCHUNK, CHUNK), jnp.int32)]
        + [pltpu.VMEM((CHUNK, D), jnp.float32) for _ in range(NBUF)]
        + [pltpu.SemaphoreType.DMA for _ in range(NBUF)]
    ),
    compiler_params=pltpu.CompilerParams(use_tc_tiling_on_sc=False),
)
def _embed_sc(idx_hbm, table_hbm, out_hbm, idx_v, *bufs_and_sems):
    bufs = bufs_and_sems[:NBUF]
    sems = bufs_and_sems[NBUF:]
    wid = lax.axis_index("s") * NC + lax.axis_index("c")
    chunk0 = wid * NCHUNK

    pltpu.sync_copy(idx_hbm.at[pl.ds(chunk0, NCHUNK)], idx_v)

    for b in range(NBUF):
        pltpu.async_copy(table_hbm.at[idx_v.at[b]], bufs[b], sems[b])

    def outer(g, carry):
        base = g * NBUF
        for b in range(NBUF):
            j = base + b
            pltpu.make_async_copy(table_hbm.at[idx_v.at[j]], bufs[b], sems[b]).wait()
            pltpu.sync_copy(bufs[b], out_hbm.at[pl.ds((chunk0 + j) * CHUNK, CHUNK)])
            pltpu.async_copy(table_hbm.at[idx_v.at[j + NBUF]], bufs[b], sems[b])
        return carry

    lax.fori_loop(0, (NCHUNK - NBUF) // NBUF, outer, 0)

    for b in range(NBUF):
        j = NCHUNK - NBUF + b
        pltpu.make_async_copy(table_hbm.at[idx_v.at[j]], bufs[b], sems[b]).wait()
        pltpu.sync_copy(bufs[b], out_hbm.at[pl.ds((chunk0 + j) * CHUNK, CHUNK)])


def kernel(token_ids, weight):
    flat = token_ids.reshape(B // CHUNK, CHUNK).astype(jnp.int32)
    out = _embed_sc(flat, weight)
    return out.reshape(token_ids.shape + (weight.shape[1],))
